# baseline (device time: 246408 ns/iter reference)
import jax
import jax.numpy as jnp
from jax import lax
from jax.experimental import pallas as pl
from jax.experimental.pallas import tpu as pltpu

N_DEV = 4
B = 4
SQ = 1024
SKV = 1024
DM = 1024
HL = 8
DH = 128
SCALE = 0.08838834764831843
BLK = 64
NEG = -1e9
BF = jnp.bfloat16


def _allgather_x(x2d):

    def body(x_ref, out_ref, send_sems, recv_sems):
        my = lax.axis_index("i")

        barrier = pltpu.get_barrier_semaphore()
        for p in range(1, N_DEV):
            peer = lax.rem(my + p, N_DEV)
            pl.semaphore_signal(barrier, inc=1, device_id=(peer,),
                                device_id_type=pl.DeviceIdType.MESH)
        pl.semaphore_wait(barrier, N_DEV - 1)

        out_ref[pl.ds(my, 1)] = x_ref[...].astype(BF)[None]

        sends = []
        for p in range(1, N_DEV):
            peer = lax.rem(my + p, N_DEV)
            rdma = pltpu.make_async_remote_copy(
                src_ref=out_ref.at[my],
                dst_ref=out_ref.at[my],
                send_sem=send_sems.at[peer],
                recv_sem=recv_sems.at[my],
                device_id=(peer,),
                device_id_type=pl.DeviceIdType.MESH,
            )
            rdma.start()
            sends.append(rdma)

        for p in range(1, N_DEV):
            j = lax.rem(my + p, N_DEV)
            recv = pltpu.make_async_remote_copy(
                src_ref=out_ref.at[my],
                dst_ref=out_ref.at[j],
                send_sem=send_sems.at[my],
                recv_sem=recv_sems.at[j],
                device_id=(j,),
                device_id_type=pl.DeviceIdType.MESH,
            )
            recv.wait_recv()
        for rdma in sends:
            rdma.wait_send()

    return pl.pallas_call(
        body,
        out_shape=jax.ShapeDtypeStruct((B, SQ, DM), BF),
        in_specs=[pl.BlockSpec(memory_space=pltpu.VMEM)],
        out_specs=pl.BlockSpec(memory_space=pltpu.VMEM),
        scratch_shapes=[
            pltpu.SemaphoreType.DMA((N_DEV,)),
            pltpu.SemaphoreType.DMA((N_DEV,)),
        ],
        compiler_params=pltpu.CompilerParams(collective_id=0),
    )(x2d)


def _compute_reduce(x_all, Wq, K_ext, V_ext, Wo):

    def body(x_ref, wq_ref, k_ref, v_ref, wo_ref, out_ref,
             k_st, v_st, q16, ctx16, wq16, wo16, bias, p_out, p_in,
             ps_sems, pr_sems, dma_sem):
        b = pl.program_id(0)
        h = pl.program_id(1)
        my = lax.axis_index("i")

        @pl.when((b == 0) & (h == 0))
        def _():
            barrier = pltpu.get_barrier_semaphore()
            for p in range(1, N_DEV):
                peer = lax.rem(my + p, N_DEV)
                pl.semaphore_signal(barrier, inc=1, device_id=(peer,),
                                    device_id_type=pl.DeviceIdType.MESH)
            pl.semaphore_wait(barrier, N_DEV - 1)

            cp = pltpu.make_async_copy(wq_ref, out_ref.at[0], dma_sem)
            cp.start()
            cp.wait()
            wq16[...] = out_ref[0].astype(BF)
            cp = pltpu.make_async_copy(wo_ref, out_ref.at[0], dma_sem)
            cp.start()
            cp.wait()
            wo16[...] = out_ref[0].astype(BF)

            r = lax.broadcasted_iota(jnp.int32, (SQ, SKV), 0) // BLK
            c = lax.broadcasted_iota(jnp.int32, (SQ, SKV), 1) // BLK
            mask = (r == c) | (c == 0) | (lax.rem(r + c, 3) == 0)
            bias[...] = jnp.where(mask, 0.0, NEG).astype(BF)

        @pl.when(h == 0)
        def _():
            cpk = pltpu.make_async_copy(
                k_ref.at[b, :, pl.ds(my * HL, HL), :], k_st, dma_sem)
            cpk.start()
            cpk.wait()
            cpv = pltpu.make_async_copy(
                v_ref.at[b, :, pl.ds(my * HL, HL), :], v_st, dma_sem)
            cpv.start()
            cpv.wait()
            q16[...] = jnp.dot(x_ref[0], wq16[...],
                               preferred_element_type=jnp.float32).astype(BF)

        k16 = k_st[:, h, :].astype(BF)
        s = lax.dot_general(q16[:, pl.ds(h * DH, DH)], k16,
                            (((1,), (1,)), ((), ())),
                            preferred_element_type=jnp.float32)
        w16 = jnp.exp((s * SCALE + bias[...]).astype(BF))
        ssum = jnp.sum(w16, axis=1, keepdims=True, dtype=jnp.float32)
        recip = (1.0 / ssum).astype(BF)
        v16 = v_st[:, h, :].astype(BF)
        ctx16[:, pl.ds(h * DH, DH)] = jnp.dot(
            w16 * recip, v16, preferred_element_type=jnp.float32).astype(BF)

        @pl.when(h == HL - 1)
        def _():
            p_out[pl.ds(b, 1)] = jnp.dot(
                ctx16[...], wo16[...],
                preferred_element_type=jnp.float32).astype(BF)[None]

            @pl.when(b != my)
            def _():
                rdma = pltpu.make_async_remote_copy(
                    src_ref=p_out.at[b],
                    dst_ref=p_in.at[my],
                    send_sem=ps_sems.at[b],
                    recv_sem=pr_sems.at[my],
                    device_id=(b,),
                    device_id_type=pl.DeviceIdType.MESH,
                )
                rdma.start()

        @pl.when((b == B - 1) & (h == HL - 1))
        def _():
            acc = p_out[pl.ds(my, 1)][0].astype(jnp.float32)
            for p in range(1, N_DEV):
                j = lax.rem(my + p, N_DEV)
                recv = pltpu.make_async_remote_copy(
                    src_ref=p_out.at[my],
                    dst_ref=p_in.at[j],
                    send_sem=ps_sems.at[my],
                    recv_sem=pr_sems.at[j],
                    device_id=(j,),
                    device_id_type=pl.DeviceIdType.MESH,
                )
                recv.wait_recv()
                acc = acc + p_in[pl.ds(j, 1)][0].astype(jnp.float32)
            out_ref[0] = acc

            for p in range(1, N_DEV):
                j = lax.rem(my + p, N_DEV)
                sent = pltpu.make_async_remote_copy(
                    src_ref=p_out.at[j],
                    dst_ref=p_in.at[my],
                    send_sem=ps_sems.at[j],
                    recv_sem=pr_sems.at[my],
                    device_id=(j,),
                    device_id_type=pl.DeviceIdType.MESH,
                )
                sent.wait_send()

    return pl.pallas_call(
        body,
        grid=(B, HL),
        in_specs=[
            pl.BlockSpec((1, SQ, DM), lambda b, h: (b, 0, 0)),
            pl.BlockSpec(memory_space=pltpu.HBM),
            pl.BlockSpec(memory_space=pltpu.HBM),
            pl.BlockSpec(memory_space=pltpu.HBM),
            pl.BlockSpec(memory_space=pltpu.HBM),
        ],
        out_specs=pl.BlockSpec(memory_space=pltpu.VMEM),
        out_shape=jax.ShapeDtypeStruct((1, SQ, DM), jnp.float32),
        scratch_shapes=[
            pltpu.VMEM((SKV, HL, DH), jnp.float32),
            pltpu.VMEM((SKV, HL, DH), jnp.float32),
            pltpu.VMEM((SQ, HL * DH), BF),
            pltpu.VMEM((SQ, HL * DH), BF),
            pltpu.VMEM((DM, HL * DH), BF),
            pltpu.VMEM((HL * DH, DM), BF),
            pltpu.VMEM((SQ, SKV), BF),
            pltpu.VMEM((N_DEV, SQ, DM), BF),
            pltpu.VMEM((N_DEV, SQ, DM), BF),
            pltpu.SemaphoreType.DMA((N_DEV,)),
            pltpu.SemaphoreType.DMA((N_DEV,)),
            pltpu.SemaphoreType.DMA,
        ],
        compiler_params=pltpu.CompilerParams(
            dimension_semantics=("arbitrary", "arbitrary"),
            collective_id=1,
            vmem_limit_bytes=56 * 1024 * 1024,
        ),
    )(x_all, Wq, K_ext, V_ext, Wo)


def kernel(x, Wq, K_ext, V_ext, Wo):
    x_all = _allgather_x(x[0])
    out = _compute_reduce(x_all, Wq, K_ext, V_ext, Wo)
    return out


# device time: 173181 ns/iter; 1.4228x vs baseline; 1.4228x over previous
import jax
import jax.numpy as jnp
from jax import lax
from jax.experimental import pallas as pl
from jax.experimental.pallas import tpu as pltpu

N_DEV = 4
B = 4
SQ = 1024
SKV = 1024
DM = 1024
HL = 8
DH = 128
SCALE = 0.08838834764831843
BLK = 64
NEG = -1e9
BF = jnp.bfloat16


def _fused(x, Wq, K_ext, V_ext, Wo):
    def body(x_ref, wq_ref, k_ref, v_ref, wo_ref, out_ref,
             x_recv, p_out, p_in, k_st, v_st, q16, ctx16, wq16, wo16, bias,
             xs_sems, xr_sems, ps_sems, pr_sems, dsem, dk_sem, dv_sem):
        b = pl.program_id(0)
        h = pl.program_id(1)
        my = lax.axis_index("i")
        bb = lax.rem(my + b, N_DEV)

        @pl.when((b == 0) & (h == 0))
        def _():
            barrier = pltpu.get_barrier_semaphore()
            for p in range(1, N_DEV):
                peer = lax.rem(my + p, N_DEV)
                pl.semaphore_signal(barrier, inc=1, device_id=(peer,),
                                    device_id_type=pl.DeviceIdType.MESH)
            pl.semaphore_wait(barrier, N_DEV - 1)

            cp = pltpu.make_async_copy(x_ref.at[0], out_ref.at[0], dsem)
            cp.start()
            cp.wait()
            x_recv[pl.ds(my, 1)] = out_ref[0].astype(BF)[None]
            for p in range(1, N_DEV):
                peer = lax.rem(my + p, N_DEV)
                rdma = pltpu.make_async_remote_copy(
                    src_ref=x_recv.at[my],
                    dst_ref=x_recv.at[my],
                    send_sem=xs_sems.at[peer],
                    recv_sem=xr_sems.at[my],
                    device_id=(peer,),
                    device_id_type=pl.DeviceIdType.MESH,
                )
                rdma.start()

            cp = pltpu.make_async_copy(wq_ref, out_ref.at[0], dsem)
            cp.start()
            cp.wait()
            wq16[...] = out_ref[0].astype(BF)
            cp = pltpu.make_async_copy(wo_ref, out_ref.at[0], dsem)
            cp.start()
            cp.wait()
            wo16[...] = out_ref[0].astype(BF)

            r = lax.broadcasted_iota(jnp.int32, (SQ, SKV), 0) // BLK
            c = lax.broadcasted_iota(jnp.int32, (SQ, SKV), 1) // BLK
            mask = (r == c) | (c == 0) | (lax.rem(r + c, 3) == 0)
            bias[...] = jnp.where(mask, 0.0, NEG).astype(BF)

        @pl.when(h == 0)
        def _():
            cpk = pltpu.make_async_copy(
                k_ref.at[bb, :, pl.ds(my * HL, HL), :], k_st, dk_sem)
            cpk.start()
            cpv = pltpu.make_async_copy(
                v_ref.at[bb, :, pl.ds(my * HL, HL), :], v_st, dv_sem)
            cpv.start()

            @pl.when(b > 0)
            def _():
                recv = pltpu.make_async_remote_copy(
                    src_ref=x_recv.at[my],
                    dst_ref=x_recv.at[bb],
                    send_sem=xs_sems.at[my],
                    recv_sem=xr_sems.at[bb],
                    device_id=(bb,),
                    device_id_type=pl.DeviceIdType.MESH,
                )
                recv.wait_recv()

            cpk.wait()
            cpv.wait()
            q16[...] = jnp.dot(x_recv[pl.ds(bb, 1)][0], wq16[...],
                               preferred_element_type=jnp.float32).astype(BF)

        k16 = k_st[:, h, :].astype(BF)
        s = lax.dot_general(q16[:, pl.ds(h * DH, DH)], k16,
                            (((1,), (1,)), ((), ())),
                            preferred_element_type=jnp.float32)
        w16 = jnp.exp((s * SCALE + bias[...]).astype(BF))
        ssum = jnp.sum(w16, axis=1, keepdims=True, dtype=jnp.float32)
        recip = (1.0 / ssum).astype(BF)
        v16 = v_st[:, h, :].astype(BF)
        ctx16[:, pl.ds(h * DH, DH)] = jnp.dot(
            w16 * recip, v16, preferred_element_type=jnp.float32).astype(BF)

        @pl.when(h == HL - 1)
        def _():
            p_out[pl.ds(bb, 1)] = jnp.dot(
                ctx16[...], wo16[...],
                preferred_element_type=jnp.float32).astype(BF)[None]

            @pl.when(b > 0)
            def _():
                rdma = pltpu.make_async_remote_copy(
                    src_ref=p_out.at[bb],
                    dst_ref=p_in.at[my],
                    send_sem=ps_sems.at[bb],
                    recv_sem=pr_sems.at[my],
                    device_id=(bb,),
                    device_id_type=pl.DeviceIdType.MESH,
                )
                rdma.start()

        @pl.when((b == B - 1) & (h == HL - 1))
        def _():
            acc = p_out[pl.ds(my, 1)][0].astype(jnp.float32)
            for p in range(1, N_DEV):
                j = lax.rem(my + N_DEV - p, N_DEV)
                recv = pltpu.make_async_remote_copy(
                    src_ref=p_out.at[my],
                    dst_ref=p_in.at[j],
                    send_sem=ps_sems.at[my],
                    recv_sem=pr_sems.at[j],
                    device_id=(j,),
                    device_id_type=pl.DeviceIdType.MESH,
                )
                recv.wait_recv()
                acc = acc + p_in[pl.ds(j, 1)][0].astype(jnp.float32)
            out_ref[0] = acc

            for p in range(1, N_DEV):
                j = lax.rem(my + p, N_DEV)
                xsent = pltpu.make_async_remote_copy(
                    src_ref=x_recv.at[my],
                    dst_ref=x_recv.at[my],
                    send_sem=xs_sems.at[j],
                    recv_sem=xr_sems.at[my],
                    device_id=(j,),
                    device_id_type=pl.DeviceIdType.MESH,
                )
                xsent.wait_send()
                psent = pltpu.make_async_remote_copy(
                    src_ref=p_out.at[j],
                    dst_ref=p_in.at[my],
                    send_sem=ps_sems.at[j],
                    recv_sem=pr_sems.at[my],
                    device_id=(j,),
                    device_id_type=pl.DeviceIdType.MESH,
                )
                psent.wait_send()

    return pl.pallas_call(
        body,
        grid=(B, HL),
        in_specs=[
            pl.BlockSpec(memory_space=pltpu.HBM),
            pl.BlockSpec(memory_space=pltpu.HBM),
            pl.BlockSpec(memory_space=pltpu.HBM),
            pl.BlockSpec(memory_space=pltpu.HBM),
            pl.BlockSpec(memory_space=pltpu.HBM),
        ],
        out_specs=pl.BlockSpec(memory_space=pltpu.VMEM),
        out_shape=jax.ShapeDtypeStruct((1, SQ, DM), jnp.float32),
        scratch_shapes=[
            pltpu.VMEM((N_DEV, SQ, DM), BF),
            pltpu.VMEM((N_DEV, SQ, DM), BF),
            pltpu.VMEM((N_DEV, SQ, DM), BF),
            pltpu.VMEM((SKV, HL, DH), jnp.float32),
            pltpu.VMEM((SKV, HL, DH), jnp.float32),
            pltpu.VMEM((SQ, HL * DH), BF),
            pltpu.VMEM((SQ, HL * DH), BF),
            pltpu.VMEM((DM, HL * DH), BF),
            pltpu.VMEM((HL * DH, DM), BF),
            pltpu.VMEM((SQ, SKV), BF),
            pltpu.SemaphoreType.DMA((N_DEV,)),
            pltpu.SemaphoreType.DMA((N_DEV,)),
            pltpu.SemaphoreType.DMA((N_DEV,)),
            pltpu.SemaphoreType.DMA((N_DEV,)),
            pltpu.SemaphoreType.DMA,
            pltpu.SemaphoreType.DMA,
            pltpu.SemaphoreType.DMA,
        ],
        compiler_params=pltpu.CompilerParams(
            dimension_semantics=("arbitrary", "arbitrary"),
            collective_id=0,
            vmem_limit_bytes=56 * 1024 * 1024,
        ),
    )(x, Wq, K_ext, V_ext, Wo)


def kernel(x, Wq, K_ext, V_ext, Wo):
    return _fused(x, Wq, K_ext, V_ext, Wo)


# device time: 154318 ns/iter; 1.5968x vs baseline; 1.1222x over previous
import jax
import jax.numpy as jnp
from jax import lax
from jax.experimental import pallas as pl
from jax.experimental.pallas import tpu as pltpu

N_DEV = 4
B = 4
SQ = 1024
SKV = 1024
DM = 1024
HL = 8
DH = 128
SCALE = 0.08838834764831843
BLK = 64
NEG = -1e9
BF = jnp.bfloat16


def _fused(x, Wq, K_ext, V_ext, Wo):
    def body(x_ref, wq_ref, k_ref, v_ref, wo_ref, out_ref,
             x_recv, p_out, p_in, k_st, v_st, q16, ctx16, wq16, wo16, bias,
             xs_sems, xr_sems, ps_sems, pr_sems, dsem, dk_sem, dv_sem):
        b = pl.program_id(0)
        h = pl.program_id(1)
        my = lax.axis_index("i")
        bb = lax.rem(my + b, N_DEV)

        @pl.when((b == 0) & (h == 0))
        def _():
            barrier = pltpu.get_barrier_semaphore()
            for p in range(1, N_DEV):
                peer = lax.rem(my + p, N_DEV)
                pl.semaphore_signal(barrier, inc=1, device_id=(peer,),
                                    device_id_type=pl.DeviceIdType.MESH)
            pl.semaphore_wait(barrier, N_DEV - 1)

            cp = pltpu.make_async_copy(x_ref.at[0], out_ref.at[0], dsem)
            cp.start()
            cp.wait()
            x_recv[pl.ds(my, 1)] = out_ref[0].astype(BF)[None]
            for p in range(1, N_DEV):
                peer = lax.rem(my + p, N_DEV)
                rdma = pltpu.make_async_remote_copy(
                    src_ref=x_recv.at[my],
                    dst_ref=x_recv.at[my],
                    send_sem=xs_sems.at[peer],
                    recv_sem=xr_sems.at[my],
                    device_id=(peer,),
                    device_id_type=pl.DeviceIdType.MESH,
                )
                rdma.start()

            cp = pltpu.make_async_copy(wq_ref, out_ref.at[0], dsem)
            cp.start()
            cp.wait()
            wq16[...] = out_ref[0].astype(BF)
            cp = pltpu.make_async_copy(wo_ref, out_ref.at[0], dsem)
            cp.start()
            cp.wait()
            wo16[...] = out_ref[0].astype(BF)

            r = lax.broadcasted_iota(jnp.int32, (SQ, SKV), 0) // BLK
            c = lax.broadcasted_iota(jnp.int32, (SQ, SKV), 1) // BLK
            mask = (r == c) | (c == 0) | (lax.rem(r + c, 3) == 0)
            bias[...] = jnp.where(mask, 0.0, NEG).astype(BF)

        @pl.when(h == 0)
        def _():
            cpk = pltpu.make_async_copy(
                k_ref.at[bb, :, pl.ds(my * HL, HL), :], k_st, dk_sem)
            cpk.start()
            cpv = pltpu.make_async_copy(
                v_ref.at[bb, :, pl.ds(my * HL, HL), :], v_st, dv_sem)
            cpv.start()

            @pl.when(b > 0)
            def _():
                recv = pltpu.make_async_remote_copy(
                    src_ref=x_recv.at[my],
                    dst_ref=x_recv.at[bb],
                    send_sem=xs_sems.at[my],
                    recv_sem=xr_sems.at[bb],
                    device_id=(bb,),
                    device_id_type=pl.DeviceIdType.MESH,
                )
                recv.wait_recv()

            cpk.wait()
            cpv.wait()
            q16[...] = jnp.dot(x_recv[pl.ds(bb, 1)][0], wq16[...],
                               preferred_element_type=jnp.float32).astype(BF)

        k16 = k_st[:, h, :].astype(BF)
        s = lax.dot_general(q16[:, pl.ds(h * DH, DH)], k16,
                            (((1,), (1,)), ((), ())),
                            preferred_element_type=jnp.float32).astype(BF)
        w16 = jnp.exp(s * jnp.bfloat16(SCALE) + bias[...])
        ssum = jnp.sum(w16, axis=1, keepdims=True, dtype=jnp.float32)
        recip = 1.0 / ssum
        v16 = v_st[:, h, :].astype(BF)
        ctx16[:, pl.ds(h * DH, DH)] = (
            jnp.dot(w16, v16, preferred_element_type=jnp.float32) * recip
        ).astype(BF)

        @pl.when(h == HL - 1)
        def _():
            p_out[pl.ds(bb, 1)] = jnp.dot(
                ctx16[...], wo16[...],
                preferred_element_type=jnp.float32).astype(BF)[None]

            @pl.when(b > 0)
            def _():
                rdma = pltpu.make_async_remote_copy(
                    src_ref=p_out.at[bb],
                    dst_ref=p_in.at[my],
                    send_sem=ps_sems.at[bb],
                    recv_sem=pr_sems.at[my],
                    device_id=(bb,),
                    device_id_type=pl.DeviceIdType.MESH,
                )
                rdma.start()

        @pl.when((b == B - 1) & (h == HL - 1))
        def _():
            acc = p_out[pl.ds(my, 1)][0].astype(jnp.float32)
            for p in range(1, N_DEV):
                j = lax.rem(my + N_DEV - p, N_DEV)
                recv = pltpu.make_async_remote_copy(
                    src_ref=p_out.at[my],
                    dst_ref=p_in.at[j],
                    send_sem=ps_sems.at[my],
                    recv_sem=pr_sems.at[j],
                    device_id=(j,),
                    device_id_type=pl.DeviceIdType.MESH,
                )
                recv.wait_recv()
                acc = acc + p_in[pl.ds(j, 1)][0].astype(jnp.float32)
            out_ref[0] = acc

            for p in range(1, N_DEV):
                j = lax.rem(my + p, N_DEV)
                xsent = pltpu.make_async_remote_copy(
                    src_ref=x_recv.at[my],
                    dst_ref=x_recv.at[my],
                    send_sem=xs_sems.at[j],
                    recv_sem=xr_sems.at[my],
                    device_id=(j,),
                    device_id_type=pl.DeviceIdType.MESH,
                )
                xsent.wait_send()
                psent = pltpu.make_async_remote_copy(
                    src_ref=p_out.at[j],
                    dst_ref=p_in.at[my],
                    send_sem=ps_sems.at[j],
                    recv_sem=pr_sems.at[my],
                    device_id=(j,),
                    device_id_type=pl.DeviceIdType.MESH,
                )
                psent.wait_send()

    return pl.pallas_call(
        body,
        grid=(B, HL),
        in_specs=[
            pl.BlockSpec(memory_space=pltpu.HBM),
            pl.BlockSpec(memory_space=pltpu.HBM),
            pl.BlockSpec(memory_space=pltpu.HBM),
            pl.BlockSpec(memory_space=pltpu.HBM),
            pl.BlockSpec(memory_space=pltpu.HBM),
        ],
        out_specs=pl.BlockSpec(memory_space=pltpu.VMEM),
        out_shape=jax.ShapeDtypeStruct((1, SQ, DM), jnp.float32),
        scratch_shapes=[
            pltpu.VMEM((N_DEV, SQ, DM), BF),
            pltpu.VMEM((N_DEV, SQ, DM), BF),
            pltpu.VMEM((N_DEV, SQ, DM), BF),
            pltpu.VMEM((SKV, HL, DH), jnp.float32),
            pltpu.VMEM((SKV, HL, DH), jnp.float32),
            pltpu.VMEM((SQ, HL * DH), BF),
            pltpu.VMEM((SQ, HL * DH), BF),
            pltpu.VMEM((DM, HL * DH), BF),
            pltpu.VMEM((HL * DH, DM), BF),
            pltpu.VMEM((SQ, SKV), BF),
            pltpu.SemaphoreType.DMA((N_DEV,)),
            pltpu.SemaphoreType.DMA((N_DEV,)),
            pltpu.SemaphoreType.DMA((N_DEV,)),
            pltpu.SemaphoreType.DMA((N_DEV,)),
            pltpu.SemaphoreType.DMA,
            pltpu.SemaphoreType.DMA,
            pltpu.SemaphoreType.DMA,
        ],
        compiler_params=pltpu.CompilerParams(
            dimension_semantics=("arbitrary", "arbitrary"),
            collective_id=0,
            vmem_limit_bytes=56 * 1024 * 1024,
        ),
    )(x, Wq, K_ext, V_ext, Wo)


def kernel(x, Wq, K_ext, V_ext, Wo):
    return _fused(x, Wq, K_ext, V_ext, Wo)


# device time: 149559 ns/iter; 1.6476x vs baseline; 1.0318x over previous
import jax
import jax.numpy as jnp
from jax import lax
from jax.experimental import pallas as pl
from jax.experimental.pallas import tpu as pltpu

N_DEV = 4
B = 4
SQ = 1024
SKV = 1024
DM = 1024
HL = 8
DH = 128
SCALE = 0.08838834764831843
BLK = 64
NEG = -1e9
BF = jnp.bfloat16


def _fused(x, Wq, K_ext, V_ext, Wo):
    def body(x_ref, wq_ref, k_ref, v_ref, wo_ref, out_ref,
             x_recv, p_out, p_in, k_st, v_st, q16, ctx16, wq16, wo16, bias,
             xs_sems, xr_sems, ps_sems, pr_sems, dsem, dk_sems, dv_sems):
        b = pl.program_id(0)
        h = pl.program_id(1)
        my = lax.axis_index("i")
        bb = lax.rem(my + b, N_DEV)

        @pl.when((b == 0) & (h == 0))
        def _():
            barrier = pltpu.get_barrier_semaphore()
            for p in range(1, N_DEV):
                peer = lax.rem(my + p, N_DEV)
                pl.semaphore_signal(barrier, inc=1, device_id=(peer,),
                                    device_id_type=pl.DeviceIdType.MESH)
            pl.semaphore_wait(barrier, N_DEV - 1)

            cp = pltpu.make_async_copy(x_ref.at[0], out_ref.at[0], dsem)
            cp.start()
            cp.wait()
            x_recv[pl.ds(my, 1)] = out_ref[0].astype(BF)[None]
            for p in range(1, N_DEV):
                peer = lax.rem(my + p, N_DEV)
                rdma = pltpu.make_async_remote_copy(
                    src_ref=x_recv.at[my],
                    dst_ref=x_recv.at[my],
                    send_sem=xs_sems.at[peer],
                    recv_sem=xr_sems.at[my],
                    device_id=(peer,),
                    device_id_type=pl.DeviceIdType.MESH,
                )
                rdma.start()

            cp = pltpu.make_async_copy(wq_ref, out_ref.at[0], dsem)
            cp.start()
            cp.wait()
            wq16[...] = out_ref[0].astype(BF)
            cp = pltpu.make_async_copy(wo_ref, out_ref.at[0], dsem)
            cp.start()
            cp.wait()
            wo16[...] = out_ref[0].astype(BF)

            r = lax.broadcasted_iota(jnp.int32, (SQ, SKV), 0) // BLK
            c = lax.broadcasted_iota(jnp.int32, (SQ, SKV), 1) // BLK
            mask = (r == c) | (c == 0) | (lax.rem(r + c, 3) == 0)
            bias[...] = jnp.where(mask, 0.0, NEG).astype(BF)

        slot = lax.rem(b, 2)

        @pl.when((b == 0) & (h == 0))
        def _():
            pltpu.make_async_copy(
                k_ref.at[bb, :, pl.ds(my * HL, HL), :],
                k_st.at[slot], dk_sems.at[slot]).start()
            pltpu.make_async_copy(
                v_ref.at[bb, :, pl.ds(my * HL, HL), :],
                v_st.at[slot], dv_sems.at[slot]).start()

        @pl.when(h == 0)
        def _():
            pltpu.make_async_copy(
                k_ref.at[bb, :, pl.ds(my * HL, HL), :],
                k_st.at[slot], dk_sems.at[slot]).wait()
            pltpu.make_async_copy(
                v_ref.at[bb, :, pl.ds(my * HL, HL), :],
                v_st.at[slot], dv_sems.at[slot]).wait()

            @pl.when(b < B - 1)
            def _():
                nb = lax.rem(bb + 1, N_DEV)
                ns = lax.rem(b + 1, 2)
                pltpu.make_async_copy(
                    k_ref.at[nb, :, pl.ds(my * HL, HL), :],
                    k_st.at[ns], dk_sems.at[ns]).start()
                pltpu.make_async_copy(
                    v_ref.at[nb, :, pl.ds(my * HL, HL), :],
                    v_st.at[ns], dv_sems.at[ns]).start()

            @pl.when(b > 0)
            def _():
                recv = pltpu.make_async_remote_copy(
                    src_ref=x_recv.at[my],
                    dst_ref=x_recv.at[bb],
                    send_sem=xs_sems.at[my],
                    recv_sem=xr_sems.at[bb],
                    device_id=(bb,),
                    device_id_type=pl.DeviceIdType.MESH,
                )
                recv.wait_recv()

            q16[...] = jnp.dot(x_recv[pl.ds(bb, 1)][0], wq16[...],
                               preferred_element_type=jnp.float32).astype(BF)

        k16 = k_st[slot, :, h, :].astype(BF)
        s = lax.dot_general(q16[:, pl.ds(h * DH, DH)], k16,
                            (((1,), (1,)), ((), ())),
                            preferred_element_type=jnp.float32).astype(BF)
        w16 = jnp.exp(s * jnp.bfloat16(SCALE) + bias[...])
        ssum = jnp.sum(w16, axis=1, keepdims=True, dtype=jnp.float32)
        recip = 1.0 / ssum
        v16 = v_st[slot, :, h, :].astype(BF)
        ctx16[:, pl.ds(h * DH, DH)] = (
            jnp.dot(w16, v16, preferred_element_type=jnp.float32) * recip
        ).astype(BF)

        @pl.when(h == HL - 1)
        def _():
            p_out[pl.ds(bb, 1)] = jnp.dot(
                ctx16[...], wo16[...],
                preferred_element_type=jnp.float32).astype(BF)[None]

            @pl.when(b > 0)
            def _():
                rdma = pltpu.make_async_remote_copy(
                    src_ref=p_out.at[bb],
                    dst_ref=p_in.at[3 - b],
                    send_sem=ps_sems.at[bb],
                    recv_sem=pr_sems.at[3 - b],
                    device_id=(bb,),
                    device_id_type=pl.DeviceIdType.MESH,
                )
                rdma.start()

        @pl.when((b == B - 1) & (h == HL - 1))
        def _():
            acc = p_out[pl.ds(my, 1)][0].astype(jnp.float32)
            for p in range(1, N_DEV):
                j = lax.rem(my + N_DEV - p, N_DEV)
                recv = pltpu.make_async_remote_copy(
                    src_ref=p_out.at[my],
                    dst_ref=p_in.at[N_DEV - 1 - p],
                    send_sem=ps_sems.at[my],
                    recv_sem=pr_sems.at[N_DEV - 1 - p],
                    device_id=(j,),
                    device_id_type=pl.DeviceIdType.MESH,
                )
                recv.wait_recv()
                acc = acc + p_in[N_DEV - 1 - p].astype(jnp.float32)
            out_ref[0] = acc

            for p in range(1, N_DEV):
                j = lax.rem(my + p, N_DEV)
                xsent = pltpu.make_async_remote_copy(
                    src_ref=x_recv.at[my],
                    dst_ref=x_recv.at[my],
                    send_sem=xs_sems.at[j],
                    recv_sem=xr_sems.at[my],
                    device_id=(j,),
                    device_id_type=pl.DeviceIdType.MESH,
                )
                xsent.wait_send()
                psent = pltpu.make_async_remote_copy(
                    src_ref=p_out.at[j],
                    dst_ref=p_in.at[0],
                    send_sem=ps_sems.at[j],
                    recv_sem=pr_sems.at[N_DEV - 1],
                    device_id=(j,),
                    device_id_type=pl.DeviceIdType.MESH,
                )
                psent.wait_send()

    return pl.pallas_call(
        body,
        grid=(B, HL),
        in_specs=[
            pl.BlockSpec(memory_space=pltpu.HBM),
            pl.BlockSpec(memory_space=pltpu.HBM),
            pl.BlockSpec(memory_space=pltpu.HBM),
            pl.BlockSpec(memory_space=pltpu.HBM),
            pl.BlockSpec(memory_space=pltpu.HBM),
        ],
        out_specs=pl.BlockSpec(memory_space=pltpu.VMEM),
        out_shape=jax.ShapeDtypeStruct((1, SQ, DM), jnp.float32),
        scratch_shapes=[
            pltpu.VMEM((N_DEV, SQ, DM), BF),
            pltpu.VMEM((N_DEV, SQ, DM), BF),
            pltpu.VMEM((N_DEV - 1, SQ, DM), BF),
            pltpu.VMEM((2, SKV, HL, DH), jnp.float32),
            pltpu.VMEM((2, SKV, HL, DH), jnp.float32),
            pltpu.VMEM((SQ, HL * DH), BF),
            pltpu.VMEM((SQ, HL * DH), BF),
            pltpu.VMEM((DM, HL * DH), BF),
            pltpu.VMEM((HL * DH, DM), BF),
            pltpu.VMEM((SQ, SKV), BF),
            pltpu.SemaphoreType.DMA((N_DEV,)),
            pltpu.SemaphoreType.DMA((N_DEV,)),
            pltpu.SemaphoreType.DMA((N_DEV,)),
            pltpu.SemaphoreType.DMA((N_DEV,)),
            pltpu.SemaphoreType.DMA,
            pltpu.SemaphoreType.DMA((2,)),
            pltpu.SemaphoreType.DMA((2,)),
        ],
        compiler_params=pltpu.CompilerParams(
            dimension_semantics=("arbitrary", "arbitrary"),
            collective_id=0,
            vmem_limit_bytes=60 * 1024 * 1024,
        ),
    )(x, Wq, K_ext, V_ext, Wo)


def kernel(x, Wq, K_ext, V_ext, Wo):
    return _fused(x, Wq, K_ext, V_ext, Wo)
